# baseline (speedup 1.0000x reference)
import functools

import jax
import jax.numpy as jnp
from jax import lax
from jax.experimental import pallas as pl
from jax.experimental.pallas import tpu as pltpu
from jax.experimental.pallas import tpu_sc as plsc

K = 9
NOISE_MIN = 0.01
NOISE_MAX = 0.5

QT = 512
MT = 1024
PAD = 128

_I32MAX = 2**31 - 1
_INF = float("inf")


CH = 128


def _k1_body(ff_ref, mbt_ref, topd_ref, topi_ref, w_ref, runv_ref, runi_ref,
             sq_ref, *, n_m):
    j = pl.program_id(1)

    @pl.when(j == 0)
    def _init():
        runv_ref[...] = jnp.full((QT, PAD), _INF, jnp.float32)
        runi_ref[...] = jnp.zeros((QT, PAD), jnp.int32)

    ff = ff_ref[...]
    mbt = mbt_ref[...]
    a2 = jnp.sum(ff * ff, axis=1, keepdims=True)
    b2 = jnp.sum(mbt * mbt, axis=0, keepdims=True)
    ab = jnp.dot(ff, mbt, preferred_element_type=jnp.float32)
    sq = a2 + b2 - 2.0 * ab
    sq_ref[...] = sq

    tau = runv_ref[...][:, 8:9]
    nch = MT // CH
    smax = 0.0
    for c in range(nch):
        blk = sq[:, c * CH:(c + 1) * CH]
        cntc = jnp.sum(jnp.where(blk < tau, 1.0, 0.0), axis=1, keepdims=True)
        smax = jnp.maximum(smax, jnp.max(cntc))

    li = lax.broadcasted_iota(jnp.int32, (QT, CH), 1)

    for k in range(K):
        @pl.when(k < smax)
        def _sweep():
            for c in range(nch):
                blk = sq_ref[:, c * CH:(c + 1) * CH]
                cm = jnp.min(blk, axis=1, keepdims=True)
                eq = blk == cm
                cil = jnp.min(jnp.where(eq, li, _I32MAX), axis=1,
                              keepdims=True)
                sq_ref[:, c * CH:(c + 1) * CH] = jnp.where(
                    eq & (li == cil), _INF, blk)
                x = cm
                xi = cil + (j * MT + c * CH)
                rv = runv_ref[...]
                ri = runi_ref[...]
                ge = rv > x
                sv = jnp.concatenate([x, rv[:, :PAD - 1]], axis=1)
                si = jnp.concatenate([xi, ri[:, :PAD - 1]], axis=1)
                sge = sv > x
                runv_ref[...] = jnp.where(ge, jnp.where(sge, sv, x), rv)
                runi_ref[...] = jnp.where(ge, jnp.where(sge, si, xi), ri)

    @pl.when(j == n_m - 1)
    def _finalize():
        rv = runv_ref[...]
        ri = runi_ref[...]
        d9 = jnp.sqrt(jnp.maximum(rv[:, :K], 1e-12))
        w9 = 1.0 / d9
        zpad = jnp.zeros((QT, PAD - K), jnp.float32)
        ipad = jnp.zeros((QT, PAD - K), jnp.int32)
        topd_ref[...] = jnp.concatenate([d9, zpad], axis=1)
        w_ref[...] = jnp.concatenate([w9, zpad], axis=1)
        topi_ref[...] = jnp.concatenate([ri[:, :K], ipad], axis=1)


def _k1(ff, mbt, interpret=False):
    bn, d = ff.shape
    m = mbt.shape[1]
    n_q, n_m = bn // QT, m // MT
    return pl.pallas_call(
        functools.partial(_k1_body, n_m=n_m),
        grid=(n_q, n_m),
        in_specs=[
            pl.BlockSpec((QT, d), lambda i, j: (i, 0)),
            pl.BlockSpec((d, MT), lambda i, j: (0, j)),
        ],
        out_specs=[
            pl.BlockSpec((QT, PAD), lambda i, j: (i, 0)),
            pl.BlockSpec((QT, PAD), lambda i, j: (i, 0)),
            pl.BlockSpec((QT, PAD), lambda i, j: (i, 0)),
        ],
        out_shape=[
            jax.ShapeDtypeStruct((bn, PAD), jnp.float32),
            jax.ShapeDtypeStruct((bn, PAD), jnp.int32),
            jax.ShapeDtypeStruct((bn, PAD), jnp.float32),
        ],
        scratch_shapes=[
            pltpu.VMEM((QT, PAD), jnp.float32),
            pltpu.VMEM((QT, PAD), jnp.int32),
            pltpu.VMEM((QT, MT), jnp.float32),
        ],
        compiler_params=pltpu.CompilerParams(
            dimension_semantics=("arbitrary", "arbitrary"),
        ),
        interpret=interpret,
    )(ff, mbt)


def _k3_body(ff_ref, v_ref, w_ref, topd_ref, iw_ref, dw_ref,
             infl_ref, noise_ref, *, bn, d):
    i = pl.program_id(0)
    ff = ff_ref[...]
    v = v_ref[...]
    w = w_ref[...]
    iw = iw_ref[...]
    dw = dw_ref[0, 0]

    s = jnp.sum(w, axis=1, keepdims=True)
    g = (ff * s - v) * (1.0 / K)
    infl = jnp.abs(g) * iw
    infl_ref[...] = infl

    mu = jnp.sum(infl, axis=1, keepdims=True) * (1.0 / d)
    diff = infl - mu
    var = jnp.sum(diff * diff, axis=1, keepdims=True) * (1.0 / (d - 1))
    inorm = diff / (jnp.sqrt(var) + 1e-8)

    topd = topd_ref[...]
    dsig_all = jnp.sum(topd, axis=1, keepdims=True) * (1.0 / K)
    dmu = jnp.sum(dsig_all) * (1.0 / bn)
    dvarnum = jnp.sum((dsig_all - dmu) ** 2)
    dstd = jnp.sqrt(dvarnum * d / (bn * d - 1))
    dsig_tile = jnp.sum(topd_ref[pl.ds(i * QT, QT), :], axis=1,
                        keepdims=True) * (1.0 / K)
    dnorm = (dsig_tile - dmu) / (dstd + 1e-8)

    comb = inorm + dw * dnorm
    noise_ref[...] = NOISE_MIN + (NOISE_MAX - NOISE_MIN) * jax.nn.sigmoid(comb)


def _k3(ff, v, w128, topd128, iw, dw, interpret=False):
    bn, d = ff.shape
    n_q = bn // QT
    return pl.pallas_call(
        functools.partial(_k3_body, bn=bn, d=d),
        grid=(n_q,),
        in_specs=[
            pl.BlockSpec((QT, d), lambda i: (i, 0)),
            pl.BlockSpec((QT, d), lambda i: (i, 0)),
            pl.BlockSpec((QT, PAD), lambda i: (i, 0)),
            pl.BlockSpec((bn, PAD), lambda i: (0, 0)),
            pl.BlockSpec((1, d), lambda i: (0, 0)),
            pl.BlockSpec(memory_space=pltpu.SMEM),
        ],
        out_specs=[
            pl.BlockSpec((QT, d), lambda i: (i, 0)),
            pl.BlockSpec((QT, d), lambda i: (i, 0)),
        ],
        out_shape=[
            jax.ShapeDtypeStruct((bn, d), jnp.float32),
            jax.ShapeDtypeStruct((bn, d), jnp.float32),
        ],
        compiler_params=pltpu.CompilerParams(
            dimension_semantics=("arbitrary",),
        ),
        interpret=interpret,
    )(ff, v, w128, topd128, iw, dw)


def _gather_v(memory_bank, idx9, w9):
    bn9 = idx9.shape[0]
    bn = bn9 // K
    d = memory_bank.shape[1]
    info = plsc.get_sparse_core_info()
    nc, ns, nl = info.num_cores, info.num_subcores, info.num_lanes
    nw = nc * ns
    qpw = bn // nw
    c = 8
    nch = qpw // c
    mesh = plsc.VectorSubcoreMesh(core_axis_name="c", subcore_axis_name="s")

    @functools.partial(
        pl.kernel,
        mesh=mesh,
        out_type=jax.ShapeDtypeStruct((bn, d), jnp.float32),
        scratch_types=[
            pltpu.VMEM((c * K,), jnp.int32),
            pltpu.VMEM((c * K, 16), jnp.float32),
            pltpu.VMEM((c * K, d), jnp.float32),
            pltpu.VMEM((c, d), jnp.float32),
            pltpu.SemaphoreType.DMA,
        ],
    )
    def k2(mb_hbm, idx_hbm, w_hbm, out_hbm, idx_v, w_v, rows_v, acc_v, sem):
        wid = lax.axis_index("s") * nc + lax.axis_index("c")

        def chunk_body(t, carry):
            q0 = wid * qpw + t * c
            pltpu.sync_copy(idx_hbm.at[pl.ds(q0 * K, c * K)], idx_v)
            pltpu.sync_copy(w_hbm.at[pl.ds(q0 * K, c * K), :], w_v)
            pltpu.async_copy(mb_hbm.at[idx_v], rows_v, sem).wait()

            def q_body(q, carry2):
                wb = [w_v[q * K + r, :] for r in range(K)]

                def j_body(jj, carry3):
                    sl = pl.ds(jj * nl, nl)
                    acc = wb[0] * rows_v[q * K + 0, sl]
                    for r in range(1, K):
                        acc = acc + wb[r] * rows_v[q * K + r, sl]
                    acc_v[q, sl] = acc
                    return carry3

                lax.fori_loop(0, d // nl, j_body, 0)
                return carry2

            lax.fori_loop(0, c, q_body, 0)
            pltpu.sync_copy(acc_v, out_hbm.at[pl.ds(q0, c)])
            return carry

        lax.fori_loop(0, nch, chunk_body, 0)

    return k2(memory_bank, idx9, w9)


def _run(features, memory_bank, influence_weight, distance_weight,
         interpret=False):
    b, n, d = features.shape
    bn = b * n
    ff = features.reshape(bn, d)
    mbt = memory_bank.T

    topd128, topi128, w128 = _k1(ff, mbt, interpret=interpret)

    idx9 = topi128[:, :K].reshape(bn * K)
    w9 = w128[:, :K].reshape(bn * K)
    w9exp = jnp.broadcast_to(w9[:, None], (bn * K, 16))
    v = _gather_v(memory_bank, idx9, w9exp)

    iw = influence_weight.reshape(1, d)
    dw = distance_weight.reshape(1, 1)
    infl, noise = _k3(ff, v, w128, topd128, iw, dw, interpret=interpret)

    topk_d = topd128[:, :K].reshape(b, n, K)
    return (infl.reshape(b, n, d), noise.reshape(b, n, d), topk_d)


def kernel(features, memory_bank, influence_weight, distance_weight):
    return _run(features, memory_bank, influence_weight, distance_weight)


# speedup vs baseline: 2.5690x; 2.5690x over previous
import functools

import jax
import jax.numpy as jnp
from jax import lax
from jax.experimental import pallas as pl
from jax.experimental.pallas import tpu as pltpu
from jax.experimental.pallas import tpu_sc as plsc

K = 9
NOISE_MIN = 0.01
NOISE_MAX = 0.5

QT = 512
MT = 1024
PAD = 128

_I32MAX = 2**31 - 1
_INF = float("inf")


CH = 128
NL = 16


def _k1_body(mb_ref, ft_ref, topd_ref, topi_ref, w_ref, runv_ref, runi_ref,
             sq_ref, *, n_m):
    j = pl.program_id(1)

    @pl.when(j == 0)
    def _init():
        runv_ref[...] = jnp.full((NL, QT), _INF, jnp.float32)
        runi_ref[...] = jnp.zeros((NL, QT), jnp.int32)

    mb = mb_ref[...]
    ft = ft_ref[...]
    a2 = jnp.sum(ft * ft, axis=0, keepdims=True)
    b2 = jnp.sum(mb * mb, axis=1, keepdims=True)
    ab = jnp.dot(mb, ft, preferred_element_type=jnp.float32)
    sq = a2 + b2 - 2.0 * ab
    sq_ref[...] = sq

    tau = runv_ref[K - 1:K, :]
    nch = MT // CH
    smax = 0.0
    for c in range(nch):
        blk = sq[c * CH:(c + 1) * CH, :]
        cntc = jnp.sum(jnp.where(blk < tau, 1.0, 0.0), axis=0, keepdims=True)
        smax = jnp.maximum(smax, jnp.max(cntc))

    li = lax.broadcasted_iota(jnp.int32, (CH, QT), 0)

    for k in range(K):
        @pl.when(k < smax)
        def _sweep():
            for c in range(nch):
                blk = sq_ref[c * CH:(c + 1) * CH, :]
                cm = jnp.min(blk, axis=0, keepdims=True)
                eq = blk == cm
                cil = jnp.min(jnp.where(eq, li, _I32MAX), axis=0,
                              keepdims=True)
                sq_ref[c * CH:(c + 1) * CH, :] = jnp.where(
                    eq & (li == cil), _INF, blk)
                x = cm
                xi = cil + (j * MT + c * CH)
                rv = runv_ref[...]
                ri = runi_ref[...]
                ge = rv > x
                sv = jnp.concatenate([x, rv[:NL - 1, :]], axis=0)
                si = jnp.concatenate([xi, ri[:NL - 1, :]], axis=0)
                sge = sv > x
                runv_ref[...] = jnp.where(ge, jnp.where(sge, sv, x), rv)
                runi_ref[...] = jnp.where(ge, jnp.where(sge, si, xi), ri)

    @pl.when(j == n_m - 1)
    def _finalize():
        rv = runv_ref[...]
        ri = runi_ref[...]
        row = lax.broadcasted_iota(jnp.int32, (NL, QT), 0)
        real = row < K
        d = jnp.sqrt(jnp.maximum(rv, 1e-12))
        topd_ref[...] = jnp.where(real, d, 0.0)
        w_ref[...] = jnp.where(real, 1.0 / d, 0.0)
        topi_ref[...] = jnp.where(real, ri, 0)


def _k1(mb, ft, interpret=False):
    m, d = mb.shape
    bn = ft.shape[1]
    n_q, n_m = bn // QT, m // MT
    return pl.pallas_call(
        functools.partial(_k1_body, n_m=n_m),
        grid=(n_q, n_m),
        in_specs=[
            pl.BlockSpec((MT, d), lambda i, j: (j, 0)),
            pl.BlockSpec((d, QT), lambda i, j: (0, i)),
        ],
        out_specs=[
            pl.BlockSpec((NL, QT), lambda i, j: (0, i)),
            pl.BlockSpec((NL, QT), lambda i, j: (0, i)),
            pl.BlockSpec((NL, QT), lambda i, j: (0, i)),
        ],
        out_shape=[
            jax.ShapeDtypeStruct((NL, bn), jnp.float32),
            jax.ShapeDtypeStruct((NL, bn), jnp.int32),
            jax.ShapeDtypeStruct((NL, bn), jnp.float32),
        ],
        scratch_shapes=[
            pltpu.VMEM((NL, QT), jnp.float32),
            pltpu.VMEM((NL, QT), jnp.int32),
            pltpu.VMEM((MT, QT), jnp.float32),
        ],
        compiler_params=pltpu.CompilerParams(
            dimension_semantics=("arbitrary", "arbitrary"),
        ),
        interpret=interpret,
    )(mb, ft)


def _k3_body(ff_ref, v_ref, w_ref, topd_ref, iw_ref, dw_ref,
             infl_ref, noise_ref, *, bn, d):
    i = pl.program_id(0)
    ff = ff_ref[...]
    v = v_ref[...]
    w = w_ref[...]
    iw = iw_ref[...]
    dw = dw_ref[0, 0]

    s = jnp.sum(w, axis=1, keepdims=True)
    g = (ff * s - v) * (1.0 / K)
    infl = jnp.abs(g) * iw
    infl_ref[...] = infl

    mu = jnp.sum(infl, axis=1, keepdims=True) * (1.0 / d)
    diff = infl - mu
    var = jnp.sum(diff * diff, axis=1, keepdims=True) * (1.0 / (d - 1))
    inorm = diff / (jnp.sqrt(var) + 1e-8)

    topd = topd_ref[...]
    dsig_all = jnp.sum(topd, axis=1, keepdims=True) * (1.0 / K)
    dmu = jnp.sum(dsig_all) * (1.0 / bn)
    dvarnum = jnp.sum((dsig_all - dmu) ** 2)
    dstd = jnp.sqrt(dvarnum * d / (bn * d - 1))
    dsig_tile = jnp.sum(topd_ref[pl.ds(i * QT, QT), :], axis=1,
                        keepdims=True) * (1.0 / K)
    dnorm = (dsig_tile - dmu) / (dstd + 1e-8)

    comb = inorm + dw * dnorm
    noise_ref[...] = NOISE_MIN + (NOISE_MAX - NOISE_MIN) * jax.nn.sigmoid(comb)


def _k3(ff, v, w128, topd128, iw, dw, interpret=False):
    bn, d = ff.shape
    n_q = bn // QT
    return pl.pallas_call(
        functools.partial(_k3_body, bn=bn, d=d),
        grid=(n_q,),
        in_specs=[
            pl.BlockSpec((QT, d), lambda i: (i, 0)),
            pl.BlockSpec((QT, d), lambda i: (i, 0)),
            pl.BlockSpec((QT, NL), lambda i: (i, 0)),
            pl.BlockSpec((bn, NL), lambda i: (0, 0)),
            pl.BlockSpec((1, d), lambda i: (0, 0)),
            pl.BlockSpec(memory_space=pltpu.SMEM),
        ],
        out_specs=[
            pl.BlockSpec((QT, d), lambda i: (i, 0)),
            pl.BlockSpec((QT, d), lambda i: (i, 0)),
        ],
        out_shape=[
            jax.ShapeDtypeStruct((bn, d), jnp.float32),
            jax.ShapeDtypeStruct((bn, d), jnp.float32),
        ],
        compiler_params=pltpu.CompilerParams(
            dimension_semantics=("arbitrary",),
        ),
        interpret=interpret,
    )(ff, v, w128, topd128, iw, dw)


def _gather_v(memory_bank, idx9, w9):
    bn9 = idx9.shape[0]
    bn = bn9 // K
    d = memory_bank.shape[1]
    info = plsc.get_sparse_core_info()
    nc, ns, nl = info.num_cores, info.num_subcores, info.num_lanes
    nw = nc * ns
    qpw = bn // nw
    c = 8
    nch = qpw // c
    mesh = plsc.VectorSubcoreMesh(core_axis_name="c", subcore_axis_name="s")

    @functools.partial(
        pl.kernel,
        mesh=mesh,
        out_type=jax.ShapeDtypeStruct((bn, d), jnp.float32),
        scratch_types=[
            pltpu.VMEM((c * K,), jnp.int32),
            pltpu.VMEM((c * K, 16), jnp.float32),
            pltpu.VMEM((c * K, d), jnp.float32),
            pltpu.VMEM((c, d), jnp.float32),
            pltpu.SemaphoreType.DMA,
        ],
    )
    def k2(mb_hbm, idx_hbm, w_hbm, out_hbm, idx_v, w_v, rows_v, acc_v, sem):
        wid = lax.axis_index("s") * nc + lax.axis_index("c")

        def chunk_body(t, carry):
            q0 = wid * qpw + t * c
            pltpu.sync_copy(idx_hbm.at[pl.ds(q0 * K, c * K)], idx_v)
            pltpu.sync_copy(w_hbm.at[pl.ds(q0 * K, c * K), :], w_v)
            pltpu.async_copy(mb_hbm.at[idx_v], rows_v, sem).wait()

            def q_body(q, carry2):
                wb = [w_v[q * K + r, :] for r in range(K)]

                def j_body(jj, carry3):
                    sl = pl.ds(jj * nl, nl)
                    acc = wb[0] * rows_v[q * K + 0, sl]
                    for r in range(1, K):
                        acc = acc + wb[r] * rows_v[q * K + r, sl]
                    acc_v[q, sl] = acc
                    return carry3

                lax.fori_loop(0, d // nl, j_body, 0)
                return carry2

            lax.fori_loop(0, c, q_body, 0)
            pltpu.sync_copy(acc_v, out_hbm.at[pl.ds(q0, c)])
            return carry

        lax.fori_loop(0, nch, chunk_body, 0)

    return k2(memory_bank, idx9, w9)


def _run(features, memory_bank, influence_weight, distance_weight,
         interpret=False):
    b, n, d = features.shape
    bn = b * n
    ff = features.reshape(bn, d)
    ft = ff.T

    topd_t, topi_t, w_t = _k1(memory_bank, ft, interpret=interpret)

    topd16 = topd_t.T
    w16 = w_t.T
    idx9 = topi_t[:K, :].T.reshape(bn * K)
    w9 = w16[:, :K].reshape(bn * K)
    w9exp = jnp.broadcast_to(w9[:, None], (bn * K, 16))
    v = _gather_v(memory_bank, idx9, w9exp)

    iw = influence_weight.reshape(1, d)
    dw = distance_weight.reshape(1, 1)
    infl, noise = _k3(ff, v, w16, topd16, iw, dw, interpret=interpret)

    topk_d = topd16[:, :K].reshape(b, n, K)
    return (infl.reshape(b, n, d), noise.reshape(b, n, d), topk_d)


def kernel(features, memory_bank, influence_weight, distance_weight):
    return _run(features, memory_bank, influence_weight, distance_weight)


# speedup vs baseline: 2.7568x; 1.0731x over previous
import functools

import jax
import jax.numpy as jnp
from jax import lax
from jax.experimental import pallas as pl
from jax.experimental.pallas import tpu as pltpu
from jax.experimental.pallas import tpu_sc as plsc

K = 9
NOISE_MIN = 0.01
NOISE_MAX = 0.5

QT = 512
MT = 1024
PAD = 128

_I32MAX = 2**31 - 1
_INF = float("inf")


CH = 128
NL = 16


def _k1_body(mb_ref, ft_ref, topd_ref, topi_ref, w_ref, runv_ref, runi_ref,
             sq_ref, *, n_m):
    j = pl.program_id(1)

    @pl.when(j == 0)
    def _init():
        runv_ref[...] = jnp.full((NL, QT), _INF, jnp.float32)
        runi_ref[...] = jnp.zeros((NL, QT), jnp.int32)

    mb = mb_ref[...]
    ft = ft_ref[...]
    a2 = jnp.sum(ft * ft, axis=0, keepdims=True)
    b2 = jnp.sum(mb * mb, axis=1, keepdims=True)
    ab = jnp.dot(mb, ft, preferred_element_type=jnp.float32)
    sq = a2 + b2 - 2.0 * ab
    sq_ref[...] = sq

    tau = runv_ref[K - 1:K, :]
    nch = MT // CH
    smax = 0.0
    for c in range(nch):
        blk = sq[c * CH:(c + 1) * CH, :]
        cntc = jnp.sum(jnp.where(blk < tau, 1.0, 0.0), axis=0, keepdims=True)
        smax = jnp.maximum(smax, jnp.max(cntc))

    li = lax.broadcasted_iota(jnp.int32, (CH, QT), 0)

    for k in range(K):
        @pl.when(k < smax)
        def _sweep():
            for c in range(nch):
                blk = sq_ref[c * CH:(c + 1) * CH, :]
                cm = jnp.min(blk, axis=0, keepdims=True)
                eq = blk == cm
                cil = jnp.min(jnp.where(eq, li, _I32MAX), axis=0,
                              keepdims=True)
                sq_ref[c * CH:(c + 1) * CH, :] = jnp.where(
                    eq & (li == cil), _INF, blk)
                x = cm
                xi = cil + (j * MT + c * CH)
                rv = runv_ref[...]
                ri = runi_ref[...]
                ge = rv > x
                sv = jnp.concatenate([x, rv[:NL - 1, :]], axis=0)
                si = jnp.concatenate([xi, ri[:NL - 1, :]], axis=0)
                sge = sv > x
                runv_ref[...] = jnp.where(ge, jnp.where(sge, sv, x), rv)
                runi_ref[...] = jnp.where(ge, jnp.where(sge, si, xi), ri)

    @pl.when(j == n_m - 1)
    def _finalize():
        rv = runv_ref[...]
        ri = runi_ref[...]
        row = lax.broadcasted_iota(jnp.int32, (NL, QT), 0)
        real = row < K
        d = jnp.sqrt(jnp.maximum(rv, 1e-12))
        topd_ref[...] = jnp.where(real, d, 0.0)
        w_ref[...] = jnp.where(real, 1.0 / d, 0.0)
        topi_ref[...] = jnp.where(real, ri, 0)


def _k1(mb, ft, interpret=False):
    m, d = mb.shape
    bn = ft.shape[1]
    n_q, n_m = bn // QT, m // MT
    return pl.pallas_call(
        functools.partial(_k1_body, n_m=n_m),
        grid=(n_q, n_m),
        in_specs=[
            pl.BlockSpec((MT, d), lambda i, j: (j, 0)),
            pl.BlockSpec((d, QT), lambda i, j: (0, i)),
        ],
        out_specs=[
            pl.BlockSpec((NL, QT), lambda i, j: (0, i)),
            pl.BlockSpec((NL, QT), lambda i, j: (0, i)),
            pl.BlockSpec((NL, QT), lambda i, j: (0, i)),
        ],
        out_shape=[
            jax.ShapeDtypeStruct((NL, bn), jnp.float32),
            jax.ShapeDtypeStruct((NL, bn), jnp.int32),
            jax.ShapeDtypeStruct((NL, bn), jnp.float32),
        ],
        scratch_shapes=[
            pltpu.VMEM((NL, QT), jnp.float32),
            pltpu.VMEM((NL, QT), jnp.int32),
            pltpu.VMEM((MT, QT), jnp.float32),
        ],
        compiler_params=pltpu.CompilerParams(
            dimension_semantics=("arbitrary", "arbitrary"),
        ),
        interpret=interpret,
    )(mb, ft)


def _k3_body(ff_ref, v_ref, w_ref, topd_ref, iw_ref, dw_ref,
             infl_ref, noise_ref, *, bn, d, tile_off):
    i = pl.program_id(0) + tile_off
    ff = ff_ref[...]
    v = v_ref[...]
    w = w_ref[...]
    iw = iw_ref[...]
    dw = dw_ref[0, 0]

    s = jnp.sum(w, axis=1, keepdims=True)
    g = (ff * s - v) * (1.0 / K)
    infl = jnp.abs(g) * iw
    infl_ref[...] = infl

    mu = jnp.sum(infl, axis=1, keepdims=True) * (1.0 / d)
    diff = infl - mu
    var = jnp.sum(diff * diff, axis=1, keepdims=True) * (1.0 / (d - 1))
    inorm = diff / (jnp.sqrt(var) + 1e-8)

    topd = topd_ref[...]
    dsig_all = jnp.sum(topd, axis=1, keepdims=True) * (1.0 / K)
    dmu = jnp.sum(dsig_all) * (1.0 / bn)
    dvarnum = jnp.sum((dsig_all - dmu) ** 2)
    dstd = jnp.sqrt(dvarnum * d / (bn * d - 1))
    dsig_tile = jnp.sum(topd_ref[pl.ds(i * QT, QT), :], axis=1,
                        keepdims=True) * (1.0 / K)
    dnorm = (dsig_tile - dmu) / (dstd + 1e-8)

    comb = inorm + dw * dnorm
    noise_ref[...] = NOISE_MIN + (NOISE_MAX - NOISE_MIN) * jax.nn.sigmoid(comb)


def _k3(ff, v, w16, topd16, iw, dw, tile_off=0, interpret=False):
    bnh, d = ff.shape
    bn = topd16.shape[0]
    n_q = bnh // QT
    return pl.pallas_call(
        functools.partial(_k3_body, bn=bn, d=d, tile_off=tile_off),
        grid=(n_q,),
        in_specs=[
            pl.BlockSpec((QT, d), lambda i: (i, 0)),
            pl.BlockSpec((QT, d), lambda i: (i, 0)),
            pl.BlockSpec((QT, NL), lambda i: (i, 0)),
            pl.BlockSpec((bn, NL), lambda i: (0, 0)),
            pl.BlockSpec((1, d), lambda i: (0, 0)),
            pl.BlockSpec(memory_space=pltpu.SMEM),
        ],
        out_specs=[
            pl.BlockSpec((QT, d), lambda i: (i, 0)),
            pl.BlockSpec((QT, d), lambda i: (i, 0)),
        ],
        out_shape=[
            jax.ShapeDtypeStruct((bnh, d), jnp.float32),
            jax.ShapeDtypeStruct((bnh, d), jnp.float32),
        ],
        compiler_params=pltpu.CompilerParams(
            dimension_semantics=("arbitrary",),
        ),
        interpret=interpret,
    )(ff, v, w16, topd16, iw, dw)


def _gather_v(memory_bank, idx9, w9):
    bn9 = idx9.shape[0]
    bn = bn9 // K
    d = memory_bank.shape[1]
    info = plsc.get_sparse_core_info()
    nc, ns, nl = info.num_cores, info.num_subcores, info.num_lanes
    nw = nc * ns
    qpw = bn // nw
    c = 8
    nch = qpw // c
    mesh = plsc.VectorSubcoreMesh(core_axis_name="c", subcore_axis_name="s")

    @functools.partial(
        pl.kernel,
        mesh=mesh,
        out_type=jax.ShapeDtypeStruct((bn, d), jnp.float32),
        scratch_types=[
            pltpu.VMEM((c * K,), jnp.int32),
            pltpu.VMEM((c * K, 16), jnp.float32),
            pltpu.VMEM((c * K, d), jnp.float32),
            pltpu.VMEM((c, d), jnp.float32),
            pltpu.SemaphoreType.DMA,
        ],
    )
    def k2(mb_hbm, idx_hbm, w_hbm, out_hbm, idx_v, w_v, rows_v, acc_v, sem):
        wid = lax.axis_index("s") * nc + lax.axis_index("c")

        def chunk_body(t, carry):
            q0 = wid * qpw + t * c
            pltpu.sync_copy(idx_hbm.at[pl.ds(q0 * K, c * K)], idx_v)
            pltpu.sync_copy(w_hbm.at[pl.ds(q0 * K, c * K), :], w_v)
            pltpu.async_copy(mb_hbm.at[idx_v], rows_v, sem).wait()

            def q_body(q, carry2):
                wb = [w_v[q * K + r, :] for r in range(K)]

                def j_body(jj, carry3):
                    sl = pl.ds(jj * nl, nl)
                    acc = wb[0] * rows_v[q * K + 0, sl]
                    for r in range(1, K):
                        acc = acc + wb[r] * rows_v[q * K + r, sl]
                    acc_v[q, sl] = acc
                    return carry3

                lax.fori_loop(0, d // nl, j_body, 0)
                return carry2

            lax.fori_loop(0, c, q_body, 0)
            pltpu.sync_copy(acc_v, out_hbm.at[pl.ds(q0, c)])
            return carry

        lax.fori_loop(0, nch, chunk_body, 0)

    return k2(memory_bank, idx9, w9)


def _run(features, memory_bank, influence_weight, distance_weight,
         interpret=False):
    b, n, d = features.shape
    bn = b * n
    ff = features.reshape(bn, d)
    ft = ff.T
    nh = 2 if (bn // 2) % QT == 0 else 1
    bnh = bn // nh

    tops, vs = [], []
    for h in range(nh):
        ft_h = ft[:, h * bnh:(h + 1) * bnh]
        topd_t, topi_t, w_t = _k1(memory_bank, ft_h, interpret=interpret)
        idx9 = topi_t[:K, :].T.reshape(bnh * K)
        w9exp = jnp.broadcast_to(
            w_t[:K, :].T.reshape(bnh * K)[:, None], (bnh * K, 16))
        vs.append(_gather_v(memory_bank, idx9, w9exp))
        tops.append((topd_t, w_t))

    topd16 = jnp.concatenate([t.T for t, _ in tops], axis=0)
    iw = influence_weight.reshape(1, d)
    dw = distance_weight.reshape(1, 1)
    infls, noises = [], []
    for h in range(nh):
        w16_h = tops[h][1].T
        infl_h, noise_h = _k3(ff[h * bnh:(h + 1) * bnh], vs[h], w16_h,
                              topd16, iw, dw, tile_off=h * (bnh // QT),
                              interpret=interpret)
        infls.append(infl_h)
        noises.append(noise_h)

    infl = jnp.concatenate(infls, axis=0)
    noise = jnp.concatenate(noises, axis=0)
    topk_d = topd16[:, :K].reshape(b, n, K)
    return (infl.reshape(b, n, d), noise.reshape(b, n, d), topk_d)


def kernel(features, memory_bank, influence_weight, distance_weight):
    return _run(features, memory_bank, influence_weight, distance_weight)


# speedup vs baseline: 2.8829x; 1.0457x over previous
import functools

import jax
import jax.numpy as jnp
from jax import lax
from jax.experimental import pallas as pl
from jax.experimental.pallas import tpu as pltpu
from jax.experimental.pallas import tpu_sc as plsc

K = 9
NOISE_MIN = 0.01
NOISE_MAX = 0.5

QT = 512
MT = 1024
PAD = 128

_I32MAX = 2**31 - 1
_INF = float("inf")


CH = 128
NL = 16


def _k1_body(mb_ref, ft_ref, topd_ref, topi_ref, w_ref, runv_ref, runi_ref,
             sq_ref, *, n_m):
    j = pl.program_id(1)

    @pl.when(j == 0)
    def _init():
        runv_ref[...] = jnp.full((NL, QT), _INF, jnp.float32)
        runi_ref[...] = jnp.zeros((NL, QT), jnp.int32)

    mb = mb_ref[...]
    ft = ft_ref[...]
    a2 = jnp.sum(ft * ft, axis=0, keepdims=True)
    b2 = jnp.sum(mb * mb, axis=1, keepdims=True)
    ab = jnp.dot(mb, ft, preferred_element_type=jnp.float32)
    sq = a2 + b2 - 2.0 * ab
    sq_ref[...] = sq

    tau = runv_ref[K - 1:K, :]
    nch = MT // CH
    smax = 0.0
    for c in range(nch):
        blk = sq[c * CH:(c + 1) * CH, :]
        cntc = jnp.sum(jnp.where(blk < tau, 1.0, 0.0), axis=0, keepdims=True)
        smax = jnp.maximum(smax, jnp.max(cntc))

    li = lax.broadcasted_iota(jnp.int32, (CH, QT), 0)

    for k in range(K):
        @pl.when(k < smax)
        def _sweep():
            for c in range(nch):
                blk = sq_ref[c * CH:(c + 1) * CH, :]
                cm = jnp.min(blk, axis=0, keepdims=True)
                eq = blk == cm
                cil = jnp.min(jnp.where(eq, li, _I32MAX), axis=0,
                              keepdims=True)
                sq_ref[c * CH:(c + 1) * CH, :] = jnp.where(
                    eq & (li == cil), _INF, blk)
                x = cm
                xi = cil + (j * MT + c * CH)
                rv = runv_ref[...]
                ri = runi_ref[...]
                ge = rv > x
                sv = jnp.concatenate([x, rv[:NL - 1, :]], axis=0)
                si = jnp.concatenate([xi, ri[:NL - 1, :]], axis=0)
                sge = sv > x
                runv_ref[...] = jnp.where(ge, jnp.where(sge, sv, x), rv)
                runi_ref[...] = jnp.where(ge, jnp.where(sge, si, xi), ri)

    @pl.when(j == n_m - 1)
    def _finalize():
        rv = runv_ref[...]
        ri = runi_ref[...]
        row = lax.broadcasted_iota(jnp.int32, (NL, QT), 0)
        real = row < K
        d = jnp.sqrt(jnp.maximum(rv, 1e-12))
        topd_ref[...] = jnp.where(real, d, 0.0)
        w_ref[...] = jnp.where(real, 1.0 / d, 0.0)
        topi_ref[...] = jnp.where(real, ri, 0)


def _k1(mb, ft, interpret=False):
    m, d = mb.shape
    bn = ft.shape[1]
    n_q, n_m = bn // QT, m // MT
    return pl.pallas_call(
        functools.partial(_k1_body, n_m=n_m),
        grid=(n_q, n_m),
        in_specs=[
            pl.BlockSpec((MT, d), lambda i, j: (j, 0)),
            pl.BlockSpec((d, QT), lambda i, j: (0, i)),
        ],
        out_specs=[
            pl.BlockSpec((NL, QT), lambda i, j: (0, i)),
            pl.BlockSpec((NL, QT), lambda i, j: (0, i)),
            pl.BlockSpec((NL, QT), lambda i, j: (0, i)),
        ],
        out_shape=[
            jax.ShapeDtypeStruct((NL, bn), jnp.float32),
            jax.ShapeDtypeStruct((NL, bn), jnp.int32),
            jax.ShapeDtypeStruct((NL, bn), jnp.float32),
        ],
        scratch_shapes=[
            pltpu.VMEM((NL, QT), jnp.float32),
            pltpu.VMEM((NL, QT), jnp.int32),
            pltpu.VMEM((MT, QT), jnp.float32),
        ],
        compiler_params=pltpu.CompilerParams(
            dimension_semantics=("arbitrary", "arbitrary"),
        ),
        interpret=interpret,
    )(mb, ft)


def _k3_body(ff_ref, v_ref, w_ref, topd_ref, iw_ref, dw_ref,
             infl_ref, noise_ref, *, bn, d, tile_off):
    i = pl.program_id(0) + tile_off
    ff = ff_ref[...]
    v = v_ref[...]
    w = w_ref[...]
    iw = iw_ref[...]
    dw = dw_ref[0, 0]

    s = jnp.sum(w, axis=1, keepdims=True)
    g = (ff * s - v) * (1.0 / K)
    infl = jnp.abs(g) * iw
    infl_ref[...] = infl

    mu = jnp.sum(infl, axis=1, keepdims=True) * (1.0 / d)
    diff = infl - mu
    var = jnp.sum(diff * diff, axis=1, keepdims=True) * (1.0 / (d - 1))
    inorm = diff / (jnp.sqrt(var) + 1e-8)

    topd = topd_ref[...]
    dsig_all = jnp.sum(topd, axis=1, keepdims=True) * (1.0 / K)
    dmu = jnp.sum(dsig_all) * (1.0 / bn)
    dvarnum = jnp.sum((dsig_all - dmu) ** 2)
    dstd = jnp.sqrt(dvarnum * d / (bn * d - 1))
    dsig_tile = jnp.sum(topd_ref[pl.ds(i * QT, QT), :], axis=1,
                        keepdims=True) * (1.0 / K)
    dnorm = (dsig_tile - dmu) / (dstd + 1e-8)

    comb = inorm + dw * dnorm
    noise_ref[...] = NOISE_MIN + (NOISE_MAX - NOISE_MIN) * jax.nn.sigmoid(comb)


def _k3(ff, v, w16, topd16, iw, dw, tile_off=0, interpret=False):
    bnh, d = ff.shape
    bn = topd16.shape[0]
    n_q = bnh // QT
    return pl.pallas_call(
        functools.partial(_k3_body, bn=bn, d=d, tile_off=tile_off),
        grid=(n_q,),
        in_specs=[
            pl.BlockSpec((QT, d), lambda i: (i, 0)),
            pl.BlockSpec((QT, d), lambda i: (i, 0)),
            pl.BlockSpec((QT, NL), lambda i: (i, 0)),
            pl.BlockSpec((bn, NL), lambda i: (0, 0)),
            pl.BlockSpec((1, d), lambda i: (0, 0)),
            pl.BlockSpec(memory_space=pltpu.SMEM),
        ],
        out_specs=[
            pl.BlockSpec((QT, d), lambda i: (i, 0)),
            pl.BlockSpec((QT, d), lambda i: (i, 0)),
        ],
        out_shape=[
            jax.ShapeDtypeStruct((bnh, d), jnp.float32),
            jax.ShapeDtypeStruct((bnh, d), jnp.float32),
        ],
        compiler_params=pltpu.CompilerParams(
            dimension_semantics=("arbitrary",),
        ),
        interpret=interpret,
    )(ff, v, w16, topd16, iw, dw)


def _gather_v(memory_bank, idx9, w9):
    bn9 = idx9.shape[0]
    bn = bn9 // K
    d = memory_bank.shape[1]
    info = plsc.get_sparse_core_info()
    nc, ns, nl = info.num_cores, info.num_subcores, info.num_lanes
    nw = nc * ns
    qpw = bn // nw
    c = 8
    nch = qpw // c
    mesh = plsc.VectorSubcoreMesh(core_axis_name="c", subcore_axis_name="s")

    @functools.partial(
        pl.kernel,
        mesh=mesh,
        out_type=jax.ShapeDtypeStruct((bn, d), jnp.float32),
        scratch_types=[
            pltpu.VMEM((c * K,), jnp.int32),
            pltpu.VMEM((c * K, 16), jnp.float32),
            pltpu.VMEM((c * K, d), jnp.float32),
            pltpu.VMEM((c, d), jnp.float32),
            pltpu.SemaphoreType.DMA,
        ],
    )
    def k2(mb_hbm, idx_hbm, w_hbm, out_hbm, idx_v, w_v, rows_v, acc_v, sem):
        wid = lax.axis_index("s") * nc + lax.axis_index("c")

        def chunk_body(t, carry):
            q0 = wid * qpw + t * c
            pltpu.sync_copy(idx_hbm.at[pl.ds(q0 * K, c * K)], idx_v)
            pltpu.sync_copy(w_hbm.at[pl.ds(q0 * K, c * K), :], w_v)
            pltpu.async_copy(mb_hbm.at[idx_v], rows_v, sem).wait()

            def q_body(q, carry2):
                wb = [w_v[q * K + r, :] for r in range(K)]

                def j_body(jj, carry3):
                    sl = pl.ds(jj * nl, nl)
                    acc = wb[0] * rows_v[q * K + 0, sl]
                    for r in range(1, K):
                        acc = acc + wb[r] * rows_v[q * K + r, sl]
                    acc_v[q, sl] = acc
                    return carry3

                lax.fori_loop(0, d // nl, j_body, 0)
                return carry2

            lax.fori_loop(0, c, q_body, 0)
            pltpu.sync_copy(acc_v, out_hbm.at[pl.ds(q0, c)])
            return carry

        lax.fori_loop(0, nch, chunk_body, 0)

    return k2(memory_bank, idx9, w9)


def _run(features, memory_bank, influence_weight, distance_weight,
         interpret=False):
    b, n, d = features.shape
    bn = b * n
    ff = features.reshape(bn, d)
    ft = ff.T
    nh = 4 if (bn // 4) % QT == 0 else (2 if (bn // 2) % QT == 0 else 1)
    bnh = bn // nh

    tops, vs = [], []
    for h in range(nh):
        ft_h = ft[:, h * bnh:(h + 1) * bnh]
        topd_t, topi_t, w_t = _k1(memory_bank, ft_h, interpret=interpret)
        idx9 = topi_t[:K, :].T.reshape(bnh * K)
        w9exp = jnp.broadcast_to(
            w_t[:K, :].T.reshape(bnh * K)[:, None], (bnh * K, 16))
        vs.append(_gather_v(memory_bank, idx9, w9exp))
        tops.append((topd_t, w_t))

    topd16 = jnp.concatenate([t.T for t, _ in tops], axis=0)
    iw = influence_weight.reshape(1, d)
    dw = distance_weight.reshape(1, 1)
    infls, noises = [], []
    for h in range(nh):
        w16_h = tops[h][1].T
        infl_h, noise_h = _k3(ff[h * bnh:(h + 1) * bnh], vs[h], w16_h,
                              topd16, iw, dw, tile_off=h * (bnh // QT),
                              interpret=interpret)
        infls.append(infl_h)
        noises.append(noise_h)

    infl = jnp.concatenate(infls, axis=0)
    noise = jnp.concatenate(noises, axis=0)
    topk_d = topd16[:, :K].reshape(b, n, K)
    return (infl.reshape(b, n, d), noise.reshape(b, n, d), topk_d)


def kernel(features, memory_bank, influence_weight, distance_weight):
    return _run(features, memory_bank, influence_weight, distance_weight)


# speedup vs baseline: 2.9779x; 1.0330x over previous
import functools

import jax
import jax.numpy as jnp
from jax import lax
from jax.experimental import pallas as pl
from jax.experimental.pallas import tpu as pltpu
from jax.experimental.pallas import tpu_sc as plsc

K = 9
NOISE_MIN = 0.01
NOISE_MAX = 0.5

QT = 512
MT = 1024
PAD = 128

_I32MAX = 2**31 - 1
_INF = float("inf")


CH = 128
NL = 16


def _k1_body(mb_ref, ft_ref, topd_ref, topi_ref, w_ref, runv_ref, runi_ref,
             sq_ref, *, n_m):
    j = pl.program_id(1)

    @pl.when(j == 0)
    def _init():
        runv_ref[...] = jnp.full((NL, QT), _INF, jnp.float32)
        runi_ref[...] = jnp.zeros((NL, QT), jnp.int32)

    mb = mb_ref[...]
    ft = ft_ref[...]
    a2 = jnp.sum(ft * ft, axis=0, keepdims=True)
    b2 = jnp.sum(mb * mb, axis=1, keepdims=True)
    ab = jnp.dot(mb, ft, preferred_element_type=jnp.float32)
    sq = a2 + b2 - 2.0 * ab
    sq_ref[...] = sq

    tau = runv_ref[K - 1:K, :]
    nch = MT // CH
    smax_c = []
    for c in range(nch):
        blk = sq[c * CH:(c + 1) * CH, :]
        cntc = jnp.sum(jnp.where(blk < tau, 1.0, 0.0), axis=0, keepdims=True)
        smax_c.append(jnp.max(cntc))

    li = lax.broadcasted_iota(jnp.int32, (CH, QT), 0)

    for k in range(K):
        for c in range(nch):
            @pl.when(k < smax_c[c])
            def _sweep(c=c):
                blk = sq_ref[c * CH:(c + 1) * CH, :]
                cm = jnp.min(blk, axis=0, keepdims=True)
                eq = blk == cm
                cil = jnp.min(jnp.where(eq, li, _I32MAX), axis=0,
                              keepdims=True)
                sq_ref[c * CH:(c + 1) * CH, :] = jnp.where(
                    eq & (li == cil), _INF, blk)
                x = cm
                xi = cil + (j * MT + c * CH)
                rv = runv_ref[...]
                ri = runi_ref[...]
                ge = rv > x
                sv = jnp.concatenate([x, rv[:NL - 1, :]], axis=0)
                si = jnp.concatenate([xi, ri[:NL - 1, :]], axis=0)
                sge = sv > x
                runv_ref[...] = jnp.where(ge, jnp.where(sge, sv, x), rv)
                runi_ref[...] = jnp.where(ge, jnp.where(sge, si, xi), ri)

    @pl.when(j == n_m - 1)
    def _finalize():
        rv = runv_ref[...]
        ri = runi_ref[...]
        row = lax.broadcasted_iota(jnp.int32, (NL, QT), 0)
        real = row < K
        d = jnp.sqrt(jnp.maximum(rv, 1e-12))
        topd_ref[...] = jnp.where(real, d, 0.0)
        w_ref[...] = jnp.where(real, 1.0 / d, 0.0)
        topi_ref[...] = jnp.where(real, ri, 0)


def _k1(mb, ft, interpret=False):
    m, d = mb.shape
    bn = ft.shape[1]
    n_q, n_m = bn // QT, m // MT
    return pl.pallas_call(
        functools.partial(_k1_body, n_m=n_m),
        grid=(n_q, n_m),
        in_specs=[
            pl.BlockSpec((MT, d), lambda i, j: (j, 0)),
            pl.BlockSpec((d, QT), lambda i, j: (0, i)),
        ],
        out_specs=[
            pl.BlockSpec((NL, QT), lambda i, j: (0, i)),
            pl.BlockSpec((NL, QT), lambda i, j: (0, i)),
            pl.BlockSpec((NL, QT), lambda i, j: (0, i)),
        ],
        out_shape=[
            jax.ShapeDtypeStruct((NL, bn), jnp.float32),
            jax.ShapeDtypeStruct((NL, bn), jnp.int32),
            jax.ShapeDtypeStruct((NL, bn), jnp.float32),
        ],
        scratch_shapes=[
            pltpu.VMEM((NL, QT), jnp.float32),
            pltpu.VMEM((NL, QT), jnp.int32),
            pltpu.VMEM((MT, QT), jnp.float32),
        ],
        compiler_params=pltpu.CompilerParams(
            dimension_semantics=("arbitrary", "arbitrary"),
        ),
        interpret=interpret,
    )(mb, ft)


def _k3_body(ff_ref, v_ref, w_ref, topd_ref, iw_ref, dw_ref,
             infl_ref, noise_ref, *, bn, d, tile_off):
    i = pl.program_id(0) + tile_off
    ff = ff_ref[...]
    v = v_ref[...]
    w = w_ref[...]
    iw = iw_ref[...]
    dw = dw_ref[0, 0]

    s = jnp.sum(w, axis=1, keepdims=True)
    g = (ff * s - v) * (1.0 / K)
    infl = jnp.abs(g) * iw
    infl_ref[...] = infl

    mu = jnp.sum(infl, axis=1, keepdims=True) * (1.0 / d)
    diff = infl - mu
    var = jnp.sum(diff * diff, axis=1, keepdims=True) * (1.0 / (d - 1))
    inorm = diff / (jnp.sqrt(var) + 1e-8)

    topd = topd_ref[...]
    dsig_all = jnp.sum(topd, axis=1, keepdims=True) * (1.0 / K)
    dmu = jnp.sum(dsig_all) * (1.0 / bn)
    dvarnum = jnp.sum((dsig_all - dmu) ** 2)
    dstd = jnp.sqrt(dvarnum * d / (bn * d - 1))
    dsig_tile = jnp.sum(topd_ref[pl.ds(i * QT, QT), :], axis=1,
                        keepdims=True) * (1.0 / K)
    dnorm = (dsig_tile - dmu) / (dstd + 1e-8)

    comb = inorm + dw * dnorm
    noise_ref[...] = NOISE_MIN + (NOISE_MAX - NOISE_MIN) * jax.nn.sigmoid(comb)


def _k3(ff, v, w16, topd16, iw, dw, tile_off=0, interpret=False):
    bnh, d = ff.shape
    bn = topd16.shape[0]
    n_q = bnh // QT
    return pl.pallas_call(
        functools.partial(_k3_body, bn=bn, d=d, tile_off=tile_off),
        grid=(n_q,),
        in_specs=[
            pl.BlockSpec((QT, d), lambda i: (i, 0)),
            pl.BlockSpec((QT, d), lambda i: (i, 0)),
            pl.BlockSpec((QT, NL), lambda i: (i, 0)),
            pl.BlockSpec((bn, NL), lambda i: (0, 0)),
            pl.BlockSpec((1, d), lambda i: (0, 0)),
            pl.BlockSpec(memory_space=pltpu.SMEM),
        ],
        out_specs=[
            pl.BlockSpec((QT, d), lambda i: (i, 0)),
            pl.BlockSpec((QT, d), lambda i: (i, 0)),
        ],
        out_shape=[
            jax.ShapeDtypeStruct((bnh, d), jnp.float32),
            jax.ShapeDtypeStruct((bnh, d), jnp.float32),
        ],
        compiler_params=pltpu.CompilerParams(
            dimension_semantics=("arbitrary",),
        ),
        interpret=interpret,
    )(ff, v, w16, topd16, iw, dw)


def _gather_v(memory_bank, idx9, w9):
    bn9 = idx9.shape[0]
    bn = bn9 // K
    d = memory_bank.shape[1]
    info = plsc.get_sparse_core_info()
    nc, ns, nl = info.num_cores, info.num_subcores, info.num_lanes
    nw = nc * ns
    qpw = bn // nw
    c = 8
    nch = qpw // c
    mesh = plsc.VectorSubcoreMesh(core_axis_name="c", subcore_axis_name="s")

    @functools.partial(
        pl.kernel,
        mesh=mesh,
        out_type=jax.ShapeDtypeStruct((bn, d), jnp.float32),
        scratch_types=[
            pltpu.VMEM((c * K,), jnp.int32),
            pltpu.VMEM((c * K, 16), jnp.float32),
            pltpu.VMEM((c * K, d), jnp.float32),
            pltpu.VMEM((c, d), jnp.float32),
            pltpu.SemaphoreType.DMA,
        ],
    )
    def k2(mb_hbm, idx_hbm, w_hbm, out_hbm, idx_v, w_v, rows_v, acc_v, sem):
        wid = lax.axis_index("s") * nc + lax.axis_index("c")

        def chunk_body(t, carry):
            q0 = wid * qpw + t * c
            pltpu.sync_copy(idx_hbm.at[pl.ds(q0 * K, c * K)], idx_v)
            pltpu.sync_copy(w_hbm.at[pl.ds(q0 * K, c * K), :], w_v)
            pltpu.async_copy(mb_hbm.at[idx_v], rows_v, sem).wait()

            def q_body(q, carry2):
                wb = [w_v[q * K + r, :] for r in range(K)]

                def j_body(jj, carry3):
                    sl = pl.ds(jj * nl, nl)
                    acc = wb[0] * rows_v[q * K + 0, sl]
                    for r in range(1, K):
                        acc = acc + wb[r] * rows_v[q * K + r, sl]
                    acc_v[q, sl] = acc
                    return carry3

                lax.fori_loop(0, d // nl, j_body, 0)
                return carry2

            lax.fori_loop(0, c, q_body, 0)
            pltpu.sync_copy(acc_v, out_hbm.at[pl.ds(q0, c)])
            return carry

        lax.fori_loop(0, nch, chunk_body, 0)

    return k2(memory_bank, idx9, w9)


def _run(features, memory_bank, influence_weight, distance_weight,
         interpret=False):
    b, n, d = features.shape
    bn = b * n
    ff = features.reshape(bn, d)
    ft = ff.T
    nh = 4 if (bn // 4) % QT == 0 else (2 if (bn // 2) % QT == 0 else 1)
    bnh = bn // nh

    tops, vs = [], []
    for h in range(nh):
        ft_h = ft[:, h * bnh:(h + 1) * bnh]
        topd_t, topi_t, w_t = _k1(memory_bank, ft_h, interpret=interpret)
        idx9 = topi_t[:K, :].T.reshape(bnh * K)
        w9exp = jnp.broadcast_to(
            w_t[:K, :].T.reshape(bnh * K)[:, None], (bnh * K, 16))
        vs.append(_gather_v(memory_bank, idx9, w9exp))
        tops.append((topd_t, w_t))

    topd16 = jnp.concatenate([t.T for t, _ in tops], axis=0)
    iw = influence_weight.reshape(1, d)
    dw = distance_weight.reshape(1, 1)
    infls, noises = [], []
    for h in range(nh):
        w16_h = tops[h][1].T
        infl_h, noise_h = _k3(ff[h * bnh:(h + 1) * bnh], vs[h], w16_h,
                              topd16, iw, dw, tile_off=h * (bnh // QT),
                              interpret=interpret)
        infls.append(infl_h)
        noises.append(noise_h)

    infl = jnp.concatenate(infls, axis=0)
    noise = jnp.concatenate(noises, axis=0)
    topk_d = topd16[:, :K].reshape(b, n, K)
    return (infl.reshape(b, n, d), noise.reshape(b, n, d), topk_d)


def kernel(features, memory_bank, influence_weight, distance_weight):
    return _run(features, memory_bank, influence_weight, distance_weight)


# speedup vs baseline: 3.0794x; 1.0341x over previous
import functools

import jax
import jax.numpy as jnp
from jax import lax
from jax.experimental import pallas as pl
from jax.experimental.pallas import tpu as pltpu
from jax.experimental.pallas import tpu_sc as plsc

K = 9
NOISE_MIN = 0.01
NOISE_MAX = 0.5

QT = 1024
MT = 1024
PAD = 128

_I32MAX = 2**31 - 1
_INF = float("inf")


CH = 128
NL = 16


def _k1_body(mb_ref, ft_ref, topd_ref, topi_ref, w_ref, runv_ref, runi_ref,
             sq_ref, *, n_m):
    j = pl.program_id(1)

    @pl.when(j == 0)
    def _init():
        runv_ref[...] = jnp.full((NL, QT), _INF, jnp.float32)
        runi_ref[...] = jnp.zeros((NL, QT), jnp.int32)

    mb = mb_ref[...]
    ft = ft_ref[...]
    a2 = jnp.sum(ft * ft, axis=0, keepdims=True)
    b2 = jnp.sum(mb * mb, axis=1, keepdims=True)
    ab = jnp.dot(mb, ft, preferred_element_type=jnp.float32)
    sq = a2 + b2 - 2.0 * ab
    sq_ref[...] = sq

    tau = runv_ref[K - 1:K, :]
    nch = MT // CH
    smax_c = []
    for c in range(nch):
        blk = sq[c * CH:(c + 1) * CH, :]
        cntc = jnp.sum(jnp.where(blk < tau, 1.0, 0.0), axis=0, keepdims=True)
        smax_c.append(jnp.max(cntc))

    li = lax.broadcasted_iota(jnp.int32, (CH, QT), 0)

    for k in range(K):
        for c in range(nch):
            @pl.when(k < smax_c[c])
            def _sweep(c=c):
                blk = sq_ref[c * CH:(c + 1) * CH, :]
                cm = jnp.min(blk, axis=0, keepdims=True)
                eq = blk == cm
                cil = jnp.min(jnp.where(eq, li, _I32MAX), axis=0,
                              keepdims=True)
                sq_ref[c * CH:(c + 1) * CH, :] = jnp.where(
                    eq & (li == cil), _INF, blk)
                x = cm
                xi = cil + (j * MT + c * CH)
                rv = runv_ref[...]
                ri = runi_ref[...]
                ge = rv > x
                sv = jnp.concatenate([x, rv[:NL - 1, :]], axis=0)
                si = jnp.concatenate([xi, ri[:NL - 1, :]], axis=0)
                sge = sv > x
                runv_ref[...] = jnp.where(ge, jnp.where(sge, sv, x), rv)
                runi_ref[...] = jnp.where(ge, jnp.where(sge, si, xi), ri)

    @pl.when(j == n_m - 1)
    def _finalize():
        rv = runv_ref[...]
        ri = runi_ref[...]
        row = lax.broadcasted_iota(jnp.int32, (NL, QT), 0)
        real = row < K
        d = jnp.sqrt(jnp.maximum(rv, 1e-12))
        topd_ref[...] = jnp.where(real, d, 0.0)
        w_ref[...] = jnp.where(real, 1.0 / d, 0.0)
        topi_ref[...] = jnp.where(real, ri, 0)


def _k1(mb, ft, interpret=False):
    m, d = mb.shape
    bn = ft.shape[1]
    n_q, n_m = bn // QT, m // MT
    return pl.pallas_call(
        functools.partial(_k1_body, n_m=n_m),
        grid=(n_q, n_m),
        in_specs=[
            pl.BlockSpec((MT, d), lambda i, j: (j, 0)),
            pl.BlockSpec((d, QT), lambda i, j: (0, i)),
        ],
        out_specs=[
            pl.BlockSpec((NL, QT), lambda i, j: (0, i)),
            pl.BlockSpec((NL, QT), lambda i, j: (0, i)),
            pl.BlockSpec((NL, QT), lambda i, j: (0, i)),
        ],
        out_shape=[
            jax.ShapeDtypeStruct((NL, bn), jnp.float32),
            jax.ShapeDtypeStruct((NL, bn), jnp.int32),
            jax.ShapeDtypeStruct((NL, bn), jnp.float32),
        ],
        scratch_shapes=[
            pltpu.VMEM((NL, QT), jnp.float32),
            pltpu.VMEM((NL, QT), jnp.int32),
            pltpu.VMEM((MT, QT), jnp.float32),
        ],
        compiler_params=pltpu.CompilerParams(
            dimension_semantics=("arbitrary", "arbitrary"),
        ),
        interpret=interpret,
    )(mb, ft)


def _k3_body(ff_ref, v_ref, w_ref, topd_ref, iw_ref, dw_ref,
             infl_ref, noise_ref, *, bn, d, tile_off):
    i = pl.program_id(0) + tile_off
    ff = ff_ref[...]
    v = v_ref[...]
    w = w_ref[...]
    iw = iw_ref[...]
    dw = dw_ref[0, 0]

    s = jnp.sum(w, axis=1, keepdims=True)
    g = (ff * s - v) * (1.0 / K)
    infl = jnp.abs(g) * iw
    infl_ref[...] = infl

    mu = jnp.sum(infl, axis=1, keepdims=True) * (1.0 / d)
    diff = infl - mu
    var = jnp.sum(diff * diff, axis=1, keepdims=True) * (1.0 / (d - 1))
    inorm = diff / (jnp.sqrt(var) + 1e-8)

    topd = topd_ref[...]
    dsig_all = jnp.sum(topd, axis=1, keepdims=True) * (1.0 / K)
    dmu = jnp.sum(dsig_all) * (1.0 / bn)
    dvarnum = jnp.sum((dsig_all - dmu) ** 2)
    dstd = jnp.sqrt(dvarnum * d / (bn * d - 1))
    dsig_tile = jnp.sum(topd_ref[pl.ds(i * QT, QT), :], axis=1,
                        keepdims=True) * (1.0 / K)
    dnorm = (dsig_tile - dmu) / (dstd + 1e-8)

    comb = inorm + dw * dnorm
    noise_ref[...] = NOISE_MIN + (NOISE_MAX - NOISE_MIN) * jax.nn.sigmoid(comb)


def _k3(ff, v, w16, topd16, iw, dw, tile_off=0, interpret=False):
    bnh, d = ff.shape
    bn = topd16.shape[0]
    n_q = bnh // QT
    return pl.pallas_call(
        functools.partial(_k3_body, bn=bn, d=d, tile_off=tile_off),
        grid=(n_q,),
        in_specs=[
            pl.BlockSpec((QT, d), lambda i: (i, 0)),
            pl.BlockSpec((QT, d), lambda i: (i, 0)),
            pl.BlockSpec((QT, NL), lambda i: (i, 0)),
            pl.BlockSpec((bn, NL), lambda i: (0, 0)),
            pl.BlockSpec((1, d), lambda i: (0, 0)),
            pl.BlockSpec(memory_space=pltpu.SMEM),
        ],
        out_specs=[
            pl.BlockSpec((QT, d), lambda i: (i, 0)),
            pl.BlockSpec((QT, d), lambda i: (i, 0)),
        ],
        out_shape=[
            jax.ShapeDtypeStruct((bnh, d), jnp.float32),
            jax.ShapeDtypeStruct((bnh, d), jnp.float32),
        ],
        compiler_params=pltpu.CompilerParams(
            dimension_semantics=("arbitrary",),
        ),
        interpret=interpret,
    )(ff, v, w16, topd16, iw, dw)


def _gather_v(memory_bank, idx9, w9):
    bn9 = idx9.shape[0]
    bn = bn9 // K
    d = memory_bank.shape[1]
    info = plsc.get_sparse_core_info()
    nc, ns, nl = info.num_cores, info.num_subcores, info.num_lanes
    nw = nc * ns
    qpw = bn // nw
    c = 8
    nch = qpw // c
    mesh = plsc.VectorSubcoreMesh(core_axis_name="c", subcore_axis_name="s")

    @functools.partial(
        pl.kernel,
        mesh=mesh,
        out_type=jax.ShapeDtypeStruct((bn, d), jnp.float32),
        scratch_types=[
            pltpu.VMEM((c * K,), jnp.int32),
            pltpu.VMEM((c * K, 16), jnp.float32),
            pltpu.VMEM((c * K, d), jnp.float32),
            pltpu.VMEM((c, d), jnp.float32),
            pltpu.SemaphoreType.DMA,
        ],
    )
    def k2(mb_hbm, idx_hbm, w_hbm, out_hbm, idx_v, w_v, rows_v, acc_v, sem):
        wid = lax.axis_index("s") * nc + lax.axis_index("c")

        def chunk_body(t, carry):
            q0 = wid * qpw + t * c
            pltpu.sync_copy(idx_hbm.at[pl.ds(q0 * K, c * K)], idx_v)
            pltpu.sync_copy(w_hbm.at[pl.ds(q0 * K, c * K), :], w_v)
            pltpu.async_copy(mb_hbm.at[idx_v], rows_v, sem).wait()

            def q_body(q, carry2):
                wb = [w_v[q * K + r, :] for r in range(K)]

                def j_body(jj, carry3):
                    sl = pl.ds(jj * nl, nl)
                    acc = wb[0] * rows_v[q * K + 0, sl]
                    for r in range(1, K):
                        acc = acc + wb[r] * rows_v[q * K + r, sl]
                    acc_v[q, sl] = acc
                    return carry3

                lax.fori_loop(0, d // nl, j_body, 0)
                return carry2

            lax.fori_loop(0, c, q_body, 0)
            pltpu.sync_copy(acc_v, out_hbm.at[pl.ds(q0, c)])
            return carry

        lax.fori_loop(0, nch, chunk_body, 0)

    return k2(memory_bank, idx9, w9)


def _run(features, memory_bank, influence_weight, distance_weight,
         interpret=False):
    b, n, d = features.shape
    bn = b * n
    ff = features.reshape(bn, d)
    ft = ff.T
    nh = 4 if (bn // 4) % QT == 0 else (2 if (bn // 2) % QT == 0 else 1)
    bnh = bn // nh

    tops, vs = [], []
    for h in range(nh):
        ft_h = ft[:, h * bnh:(h + 1) * bnh]
        topd_t, topi_t, w_t = _k1(memory_bank, ft_h, interpret=interpret)
        idx9 = topi_t[:K, :].T.reshape(bnh * K)
        w9exp = jnp.broadcast_to(
            w_t[:K, :].T.reshape(bnh * K)[:, None], (bnh * K, 16))
        vs.append(_gather_v(memory_bank, idx9, w9exp))
        tops.append((topd_t, w_t))

    topd16 = jnp.concatenate([t.T for t, _ in tops], axis=0)
    iw = influence_weight.reshape(1, d)
    dw = distance_weight.reshape(1, 1)
    infls, noises = [], []
    for h in range(nh):
        w16_h = tops[h][1].T
        infl_h, noise_h = _k3(ff[h * bnh:(h + 1) * bnh], vs[h], w16_h,
                              topd16, iw, dw, tile_off=h * (bnh // QT),
                              interpret=interpret)
        infls.append(infl_h)
        noises.append(noise_h)

    infl = jnp.concatenate(infls, axis=0)
    noise = jnp.concatenate(noises, axis=0)
    topk_d = topd16[:, :K].reshape(b, n, K)
    return (infl.reshape(b, n, d), noise.reshape(b, n, d), topk_d)


def kernel(features, memory_bank, influence_weight, distance_weight):
    return _run(features, memory_bank, influence_weight, distance_weight)
